# baseline (speedup 1.0000x reference)
import functools

import jax
import jax.numpy as jnp
from jax import lax
from jax.experimental import pallas as pl
from jax.experimental.pallas import tpu as pltpu
from jax.experimental.pallas import tpu_sc as plsc

VOCAB = 100000
HIDDEN = 1024
MAX_POS = 2048
N_TYPES = 2
BATCH = 4
SEQ = 2048
EPS = 1e-5

NC = 2
NS = 16
NW = NC * NS
NTOK = BATCH * SEQ
TOK = NTOK // NW
POSW = SEQ // NW
CH = 16
NCHUNK = TOK // CH
G = 8
NG = CH // G
LANES = 16
NVEC = HIDDEN // LANES
NSLOT = 3

_f32 = jnp.float32


def _rsqrt_vec(xv):
    iv = lax.bitcast_convert_type(xv, jnp.int32)
    iv = jnp.int32(0x5F3759DF) - lax.shift_right_logical(iv, 1)
    y = lax.bitcast_convert_type(iv, _f32)
    for _ in range(3):
        y = y * (_f32(1.5) - _f32(0.5) * xv * y * y)
    return y


def _body(ids_hbm, pids_hbm, tids_hbm, word_hbm, pos_hbm, type_hbm,
          gamma_hbm, beta_hbm, out_hbm,
          ids_v, pids_v, tids_v, posres_v, rows_v, ttab_v, dif_v,
          gam_v, bet_v, gsem0, gsem1, gsem2, osem0, osem1, osem2):
    c = lax.axis_index("c")
    s = lax.axis_index("s")
    wid = c * NS + s
    pbase = wid * POSW

    for b in range(BATCH):
        pltpu.sync_copy(ids_hbm.at[pl.ds(b * SEQ + pbase, POSW)],
                        ids_v.at[pl.ds(b * POSW, POSW)])
        pltpu.sync_copy(tids_hbm.at[pl.ds(b * SEQ + pbase, POSW)],
                        tids_v.at[pl.ds(b * POSW, POSW)])
    pltpu.sync_copy(pids_hbm.at[pl.ds(pbase, POSW)], pids_v)
    pltpu.sync_copy(gamma_hbm, gam_v)
    pltpu.sync_copy(beta_hbm, bet_v)
    pltpu.sync_copy(type_hbm, ttab_v)
    pltpu.async_copy(pos_hbm.at[pids_v], posres_v, gsem0).wait()

    def mk_dif(j, _):
        h = pl.ds(j * LANES, LANES)
        dif_v[h] = ttab_v[1, h] - ttab_v[0, h]
        return 0

    lax.fori_loop(0, NVEC, mk_dif, 0)

    inv_h = _f32(1.0 / HIDDEN)
    gsem = (gsem0, gsem1, gsem2)
    osem = (osem0, osem1, osem2)

    def compute(ck, sl):
        seg = lax.rem(ck, 4)

        def group_body(g, _):
            tb = g * G
            tf = []
            for u in range(G):
                tid = plsc.load_gather(
                    tids_v, [jnp.full((LANES,), ck * CH + tb + u, jnp.int32)])
                tf.append(tid.astype(_f32))

            def p1(j, carry):
                sms, sss = carry
                h = pl.ds(j * LANES, LANES)
                t0 = ttab_v[0, h]
                df = dif_v[h]
                nsm = []
                nss = []
                for u in range(G):
                    x = rows_v[sl, tb + u, h] + posres_v[seg * CH + tb + u, h]
                    x = x + (tf[u] * df + t0)
                    rows_v[sl, tb + u, h] = x
                    nsm.append(sms[u] + x)
                    nss.append(sss[u] + x * x)
                return (tuple(nsm), tuple(nss))

            z = jnp.zeros((LANES,), _f32)
            zs = (z,) * G
            sms, sss = plsc.parallel_loop(0, NVEC, unroll=4,
                                          carry=(zs, zs))(p1)

            rstd = []
            shift = []
            for u in range(G):
                mean = jnp.sum(sms[u]) * inv_h
                var = jnp.sum(sss[u]) * inv_h - mean * mean
                r = _rsqrt_vec(jnp.full((LANES,), var + _f32(EPS), _f32))
                rstd.append(r)
                shift.append((-mean) * r)

            @plsc.parallel_loop(0, NVEC, unroll=4)
            def p2(j):
                h = pl.ds(j * LANES, LANES)
                gj = gam_v[h]
                bj = bet_v[h]
                for u in range(G):
                    x = rows_v[sl, tb + u, h]
                    rows_v[sl, tb + u, h] = (x * rstd[u] + shift[u]) * gj + bj

            return 0

        lax.fori_loop(0, NG, group_body, 0)

    def issue_gather(ck, sl):
        pltpu.async_copy(
            word_hbm.at[ids_v.at[pl.ds(ck * CH, CH)]], rows_v.at[sl],
            gsem[sl])

    def wait_gather(sl):
        pltpu.make_async_copy(
            word_hbm.at[ids_v.at[pl.ds(0, CH)]], rows_v.at[sl],
            gsem[sl]).wait()

    def issue_out(ck, sl):
        b = lax.div(ck, 4)
        seg = lax.rem(ck, 4)
        pltpu.async_copy(
            rows_v.at[sl],
            out_hbm.at[pl.ds(b * SEQ + pbase + seg * CH, CH)], osem[sl])

    def wait_out(sl):
        pltpu.make_async_copy(
            rows_v.at[sl], out_hbm.at[pl.ds(0, CH)], osem[sl]).wait()

    issue_gather(0, 0)
    issue_gather(1, 1)

    def pipe_body(k, _):
        for i in range(NSLOT):
            ck = k * NSLOT + i
            wait_gather(i)
            compute(ck, i)
            issue_out(ck, i)
            nsl = (i + 2) % NSLOT

            @pl.when(ck + 2 < NCHUNK)
            def _refill():
                @pl.when(ck >= 1)
                def _drain_out():
                    wait_out(nsl)

                issue_gather(ck + 2, nsl)
        return 0

    lax.fori_loop(0, (NCHUNK - 1) // NSLOT, pipe_body, 0)

    last = NCHUNK - 1
    lsl = last % NSLOT
    wait_gather(lsl)
    compute(jnp.int32(last), lsl)
    issue_out(jnp.int32(last), lsl)
    for sl in range(NSLOT):
        wait_out(sl)


@jax.jit
def _embed_ln(ids, pids, tids, word_table, pos_table, type_table, gamma, beta):
    mesh = plsc.VectorSubcoreMesh(core_axis_name="c", subcore_axis_name="s")
    k = functools.partial(
        pl.kernel,
        mesh=mesh,
        compiler_params=pltpu.CompilerParams(needs_layout_passes=False),
        out_type=jax.ShapeDtypeStruct((NTOK, HIDDEN), _f32),
        scratch_types=[
            pltpu.VMEM((TOK,), jnp.int32),
            pltpu.VMEM((POSW,), jnp.int32),
            pltpu.VMEM((TOK,), jnp.int32),
            pltpu.VMEM((POSW, HIDDEN), _f32),
            pltpu.VMEM((NSLOT, CH, HIDDEN), _f32),
            pltpu.VMEM((N_TYPES, HIDDEN), _f32),
            pltpu.VMEM((HIDDEN,), _f32),
            pltpu.VMEM((HIDDEN,), _f32),
            pltpu.VMEM((HIDDEN,), _f32),
            pltpu.SemaphoreType.DMA,
            pltpu.SemaphoreType.DMA,
            pltpu.SemaphoreType.DMA,
            pltpu.SemaphoreType.DMA,
            pltpu.SemaphoreType.DMA,
            pltpu.SemaphoreType.DMA,
        ],
    )(_body)
    return k(ids, pids, tids, word_table, pos_table, type_table, gamma, beta)


def kernel(input_ids, position_ids, type_token_ids, word_table, pos_table,
           type_table, gamma, beta):
    ids = input_ids.reshape(NTOK).astype(jnp.int32)
    pids = position_ids.reshape(SEQ).astype(jnp.int32)
    tids = type_token_ids.reshape(NTOK).astype(jnp.int32)
    out = _embed_ln(ids, pids, tids, word_table, pos_table, type_table,
                    gamma, beta)
    return out.reshape(BATCH, SEQ, HIDDEN)


# speedup vs baseline: 1.0692x; 1.0692x over previous
import functools

import jax
import jax.numpy as jnp
from jax import lax
from jax.experimental import pallas as pl
from jax.experimental.pallas import tpu as pltpu
from jax.experimental.pallas import tpu_sc as plsc

VOCAB = 100000
HIDDEN = 1024
MAX_POS = 2048
N_TYPES = 2
BATCH = 4
SEQ = 2048
EPS = 1e-5

NC = 2
NS = 16
NW = NC * NS
NTOK = BATCH * SEQ
TOK = NTOK // NW
POSW = SEQ // NW
CH = 16
NCHUNK = TOK // CH
G = 8
NG = CH // G
LANES = 16
NVEC = HIDDEN // LANES
NSLOT = 3

_f32 = jnp.float32


def _rsqrt_vec(xv):
    iv = lax.bitcast_convert_type(xv, jnp.int32)
    iv = jnp.int32(0x5F3759DF) - lax.shift_right_logical(iv, 1)
    y = lax.bitcast_convert_type(iv, _f32)
    for _ in range(3):
        y = y * (_f32(1.5) - _f32(0.5) * xv * y * y)
    return y


def _body(ids_hbm, pids_hbm, tids_hbm, word_hbm, pos_hbm, type_hbm,
          gamma_hbm, beta_hbm, out_hbm,
          ids_v, pids_v, tids_v, posres_v, rows_v, ttab_v, dif_v,
          gam_v, bet_v, gsem0, gsem1, gsem2, osem0, osem1, osem2):
    c = lax.axis_index("c")
    s = lax.axis_index("s")
    wid = c * NS + s
    pbase = wid * POSW

    inv_h = _f32(1.0 / HIDDEN)
    gsem = (gsem0, gsem1, gsem2)
    osem = (osem0, osem1, osem2)

    for b in range(BATCH):
        pltpu.sync_copy(ids_hbm.at[pl.ds(b * SEQ + pbase, POSW)],
                        ids_v.at[pl.ds(b * POSW, POSW)])
    pltpu.async_copy(
        word_hbm.at[ids_v.at[pl.ds(0, CH)]], rows_v.at[0], gsem0)
    pltpu.async_copy(
        word_hbm.at[ids_v.at[pl.ds(CH, CH)]], rows_v.at[1], gsem1)

    for b in range(BATCH):
        pltpu.sync_copy(tids_hbm.at[pl.ds(b * SEQ + pbase, POSW)],
                        tids_v.at[pl.ds(b * POSW, POSW)])
    pltpu.sync_copy(pids_hbm.at[pl.ds(pbase, POSW)], pids_v)
    posres_copy = pltpu.async_copy(pos_hbm.at[pids_v], posres_v, osem0)
    pltpu.sync_copy(gamma_hbm, gam_v)
    pltpu.sync_copy(beta_hbm, bet_v)
    pltpu.sync_copy(type_hbm, ttab_v)

    @plsc.parallel_loop(0, NVEC, unroll=2)
    def mk_dif(j):
        h = pl.ds(j * LANES, LANES)
        dif_v[h] = ttab_v[1, h] - ttab_v[0, h]

    posres_copy.wait()

    def compute(ck, sl):
        seg = lax.rem(ck, 4)

        def group_body(g, _):
            tb = g * G
            tf = []
            for u in range(G):
                tid = plsc.load_gather(
                    tids_v, [jnp.full((LANES,), ck * CH + tb + u, jnp.int32)])
                tf.append(tid.astype(_f32))

            def p1(j, carry):
                sms, sss = carry
                h = pl.ds(j * LANES, LANES)
                t0 = ttab_v[0, h]
                df = dif_v[h]
                nsm = []
                nss = []
                for u in range(G):
                    x = rows_v[sl, tb + u, h] + posres_v[seg * CH + tb + u, h]
                    x = x + (tf[u] * df + t0)
                    rows_v[sl, tb + u, h] = x
                    nsm.append(sms[u] + x)
                    nss.append(sss[u] + x * x)
                return (tuple(nsm), tuple(nss))

            z = jnp.zeros((LANES,), _f32)
            zs = (z,) * G
            sms, sss = plsc.parallel_loop(0, NVEC, unroll=2,
                                          carry=(zs, zs))(p1)

            rstd = []
            shift = []
            for u in range(G):
                mean = jnp.sum(sms[u]) * inv_h
                var = jnp.sum(sss[u]) * inv_h - mean * mean
                r = _rsqrt_vec(jnp.full((LANES,), var + _f32(EPS), _f32))
                rstd.append(r)
                shift.append((-mean) * r)

            @plsc.parallel_loop(0, NVEC, unroll=2)
            def p2(j):
                h = pl.ds(j * LANES, LANES)
                gj = gam_v[h]
                bj = bet_v[h]
                for u in range(G):
                    x = rows_v[sl, tb + u, h]
                    rows_v[sl, tb + u, h] = (x * rstd[u] + shift[u]) * gj + bj

            return 0

        lax.fori_loop(0, NG, group_body, 0)

    def issue_gather(ck, sl):
        pltpu.async_copy(
            word_hbm.at[ids_v.at[pl.ds(ck * CH, CH)]], rows_v.at[sl],
            gsem[sl])

    def wait_gather(sl):
        pltpu.make_async_copy(
            word_hbm.at[ids_v.at[pl.ds(0, CH)]], rows_v.at[sl],
            gsem[sl]).wait()

    def issue_out(ck, sl):
        b = lax.div(ck, 4)
        seg = lax.rem(ck, 4)
        pltpu.async_copy(
            rows_v.at[sl],
            out_hbm.at[pl.ds(b * SEQ + pbase + seg * CH, CH)], osem[sl])

    def wait_out(sl):
        pltpu.make_async_copy(
            rows_v.at[sl], out_hbm.at[pl.ds(0, CH)], osem[sl]).wait()

    def pipe_body(k, _):
        for i in range(NSLOT):
            ck = k * NSLOT + i
            wait_gather(i)
            compute(ck, i)
            issue_out(ck, i)
            nsl = (i + 2) % NSLOT

            @pl.when(ck + 2 < NCHUNK)
            def _refill():
                @pl.when(ck >= 1)
                def _drain_out():
                    wait_out(nsl)

                issue_gather(ck + 2, nsl)
        return 0

    lax.fori_loop(0, (NCHUNK - 1) // NSLOT, pipe_body, 0)

    last = NCHUNK - 1
    lsl = last % NSLOT
    wait_gather(lsl)
    compute(jnp.int32(last), lsl)
    issue_out(jnp.int32(last), lsl)
    for sl in range(NSLOT):
        wait_out(sl)


@jax.jit
def _embed_ln(ids, pids, tids, word_table, pos_table, type_table, gamma, beta):
    mesh = plsc.VectorSubcoreMesh(core_axis_name="c", subcore_axis_name="s")
    k = functools.partial(
        pl.kernel,
        mesh=mesh,
        compiler_params=pltpu.CompilerParams(needs_layout_passes=False),
        out_type=jax.ShapeDtypeStruct((NTOK, HIDDEN), _f32),
        scratch_types=[
            pltpu.VMEM((TOK,), jnp.int32),
            pltpu.VMEM((POSW,), jnp.int32),
            pltpu.VMEM((TOK,), jnp.int32),
            pltpu.VMEM((POSW, HIDDEN), _f32),
            pltpu.VMEM((NSLOT, CH, HIDDEN), _f32),
            pltpu.VMEM((N_TYPES, HIDDEN), _f32),
            pltpu.VMEM((HIDDEN,), _f32),
            pltpu.VMEM((HIDDEN,), _f32),
            pltpu.VMEM((HIDDEN,), _f32),
            pltpu.SemaphoreType.DMA,
            pltpu.SemaphoreType.DMA,
            pltpu.SemaphoreType.DMA,
            pltpu.SemaphoreType.DMA,
            pltpu.SemaphoreType.DMA,
            pltpu.SemaphoreType.DMA,
        ],
    )(_body)
    return k(ids, pids, tids, word_table, pos_table, type_table, gamma, beta)


def kernel(input_ids, position_ids, type_token_ids, word_table, pos_table,
           type_table, gamma, beta):
    ids = input_ids.reshape(NTOK).astype(jnp.int32)
    pids = position_ids.reshape(SEQ).astype(jnp.int32)
    tids = type_token_ids.reshape(NTOK).astype(jnp.int32)
    out = _embed_ln(ids, pids, tids, word_table, pos_table, type_table,
                    gamma, beta)
    return out.reshape(BATCH, SEQ, HIDDEN)


# speedup vs baseline: 1.0912x; 1.0206x over previous
import functools

import jax
import jax.numpy as jnp
from jax import lax
from jax.experimental import pallas as pl
from jax.experimental.pallas import tpu as pltpu
from jax.experimental.pallas import tpu_sc as plsc

VOCAB = 100000
HIDDEN = 1024
MAX_POS = 2048
N_TYPES = 2
BATCH = 4
SEQ = 2048
EPS = 1e-5

NC = 2
NS = 16
NW = NC * NS
NTOK = BATCH * SEQ
TOK = NTOK // NW
POSW = SEQ // NW
CH = 16
NCHUNK = TOK // CH
G = 8
NG = CH // G
LANES = 16
NVEC = HIDDEN // LANES
NSLOT = 3

_f32 = jnp.float32


def _rsqrt_vec(xv):
    iv = lax.bitcast_convert_type(xv, jnp.int32)
    iv = jnp.int32(0x5F3759DF) - lax.shift_right_logical(iv, 1)
    y = lax.bitcast_convert_type(iv, _f32)
    for _ in range(3):
        y = y * (_f32(1.5) - _f32(0.5) * xv * y * y)
    return y


def _body(ids_hbm, pids_hbm, tids_hbm, word_hbm, pos_hbm, type_hbm,
          gamma_hbm, beta_hbm, out_hbm,
          ids_v, pids_v, tids_v, posres_v, rows_v, ttab_v, dif_v,
          gam_v, bet_v, gsem0, gsem1, gsem2, osem0, osem1, osem2):
    c = lax.axis_index("c")
    s = lax.axis_index("s")
    wid = c * NS + s
    pbase = wid * POSW

    inv_h = _f32(1.0 / HIDDEN)
    gsem = (gsem0, gsem1, gsem2)
    osem = (osem0, osem1, osem2)

    pltpu.sync_copy(pids_hbm.at[pl.ds(pbase, POSW)], pids_v)
    posres_copy = pltpu.async_copy(pos_hbm.at[pids_v], posres_v, osem0)
    for b in range(BATCH):
        pltpu.sync_copy(ids_hbm.at[pl.ds(b * SEQ + pbase, POSW)],
                        ids_v.at[pl.ds(b * POSW, POSW)])
    pltpu.async_copy(
        word_hbm.at[ids_v.at[pl.ds(0, CH)]], rows_v.at[0], gsem0)
    pltpu.async_copy(
        word_hbm.at[ids_v.at[pl.ds(CH, CH)]], rows_v.at[1], gsem1)

    for b in range(BATCH):
        pltpu.sync_copy(tids_hbm.at[pl.ds(b * SEQ + pbase, POSW)],
                        tids_v.at[pl.ds(b * POSW, POSW)])
    pltpu.sync_copy(gamma_hbm, gam_v)
    pltpu.sync_copy(beta_hbm, bet_v)
    pltpu.sync_copy(type_hbm, ttab_v)

    @plsc.parallel_loop(0, NVEC, unroll=2)
    def mk_dif(j):
        h = pl.ds(j * LANES, LANES)
        dif_v[h] = ttab_v[1, h] - ttab_v[0, h]

    posres_copy.wait()

    def compute(ck, sl):
        seg = lax.rem(ck, 4)

        def group_body(g, _):
            tb = g * G
            tf = []
            for u in range(G):
                tid = plsc.load_gather(
                    tids_v, [jnp.full((LANES,), ck * CH + tb + u, jnp.int32)])
                tf.append(tid.astype(_f32))

            def p1(j, carry):
                sms, sss = carry
                h = pl.ds(j * LANES, LANES)
                t0 = ttab_v[0, h]
                df = dif_v[h]
                nsm = []
                nss = []
                for u in range(G):
                    x = rows_v[sl, tb + u, h] + posres_v[seg * CH + tb + u, h]
                    x = x + (tf[u] * df + t0)
                    rows_v[sl, tb + u, h] = x
                    nsm.append(sms[u] + x)
                    nss.append(sss[u] + x * x)
                return (tuple(nsm), tuple(nss))

            z = jnp.zeros((LANES,), _f32)
            zs = (z,) * G
            sms, sss = plsc.parallel_loop(0, NVEC, unroll=2,
                                          carry=(zs, zs))(p1)

            rstd = []
            shift = []
            for u in range(G):
                mean = jnp.sum(sms[u]) * inv_h
                var = jnp.sum(sss[u]) * inv_h - mean * mean
                r = _rsqrt_vec(jnp.full((LANES,), var + _f32(EPS), _f32))
                rstd.append(r)
                shift.append((-mean) * r)

            @plsc.parallel_loop(0, NVEC, unroll=2)
            def p2(j):
                h = pl.ds(j * LANES, LANES)
                gj = gam_v[h]
                bj = bet_v[h]
                for u in range(G):
                    x = rows_v[sl, tb + u, h]
                    rows_v[sl, tb + u, h] = (x * rstd[u] + shift[u]) * gj + bj

            return 0

        lax.fori_loop(0, NG, group_body, 0)

    def issue_gather(ck, sl):
        pltpu.async_copy(
            word_hbm.at[ids_v.at[pl.ds(ck * CH, CH)]], rows_v.at[sl],
            gsem[sl])

    def wait_gather(sl):
        pltpu.make_async_copy(
            word_hbm.at[ids_v.at[pl.ds(0, CH)]], rows_v.at[sl],
            gsem[sl]).wait()

    def issue_out(ck, sl):
        b = lax.div(ck, 4)
        seg = lax.rem(ck, 4)
        pltpu.async_copy(
            rows_v.at[sl],
            out_hbm.at[pl.ds(b * SEQ + pbase + seg * CH, CH)], osem[sl])

    def wait_out(sl):
        pltpu.make_async_copy(
            rows_v.at[sl], out_hbm.at[pl.ds(0, CH)], osem[sl]).wait()

    def pipe_body(k, _):
        for i in range(NSLOT):
            ck = k * NSLOT + i
            wait_gather(i)
            compute(ck, i)
            issue_out(ck, i)
            nsl = (i + 2) % NSLOT

            @pl.when(ck + 2 < NCHUNK)
            def _refill():
                @pl.when(ck >= 1)
                def _drain_out():
                    wait_out(nsl)

                issue_gather(ck + 2, nsl)
        return 0

    lax.fori_loop(0, (NCHUNK - 1) // NSLOT, pipe_body, 0)

    last = NCHUNK - 1
    lsl = last % NSLOT
    wait_gather(lsl)
    compute(jnp.int32(last), lsl)
    issue_out(jnp.int32(last), lsl)
    for sl in range(NSLOT):
        wait_out(sl)


@jax.jit
def _embed_ln(ids, pids, tids, word_table, pos_table, type_table, gamma, beta):
    mesh = plsc.VectorSubcoreMesh(core_axis_name="c", subcore_axis_name="s")
    k = functools.partial(
        pl.kernel,
        mesh=mesh,
        compiler_params=pltpu.CompilerParams(needs_layout_passes=False),
        out_type=jax.ShapeDtypeStruct((NTOK, HIDDEN), _f32),
        scratch_types=[
            pltpu.VMEM((TOK,), jnp.int32),
            pltpu.VMEM((POSW,), jnp.int32),
            pltpu.VMEM((TOK,), jnp.int32),
            pltpu.VMEM((POSW, HIDDEN), _f32),
            pltpu.VMEM((NSLOT, CH, HIDDEN), _f32),
            pltpu.VMEM((N_TYPES, HIDDEN), _f32),
            pltpu.VMEM((HIDDEN,), _f32),
            pltpu.VMEM((HIDDEN,), _f32),
            pltpu.VMEM((HIDDEN,), _f32),
            pltpu.SemaphoreType.DMA,
            pltpu.SemaphoreType.DMA,
            pltpu.SemaphoreType.DMA,
            pltpu.SemaphoreType.DMA,
            pltpu.SemaphoreType.DMA,
            pltpu.SemaphoreType.DMA,
        ],
    )(_body)
    return k(ids, pids, tids, word_table, pos_table, type_table, gamma, beta)


def kernel(input_ids, position_ids, type_token_ids, word_table, pos_table,
           type_table, gamma, beta):
    ids = input_ids.reshape(NTOK).astype(jnp.int32)
    pids = position_ids.reshape(SEQ).astype(jnp.int32)
    tids = type_token_ids.reshape(NTOK).astype(jnp.int32)
    out = _embed_ln(ids, pids, tids, word_table, pos_table, type_table,
                    gamma, beta)
    return out.reshape(BATCH, SEQ, HIDDEN)
